# baseline (speedup 1.0000x reference)
import functools

import jax
import jax.numpy as jnp
from jax import lax
from jax.experimental import pallas as pl
from jax.experimental.pallas import tpu as pltpu

_K = 1024
_D = 256
_ROWS = 256


def _tc_body(x_ref, f2_ref, e2_ref, emb_ref, tgt_ref, q_ref, loss_ref):
    i = pl.program_id(0)
    x = x_ref[...]
    emb = emb_ref[...]
    scores = lax.dot_general(
        x, emb, (((1,), (1,)), ((), ())),
        preferred_element_type=jnp.float32)
    t = f2_ref[...] + e2_ref[...] - 2.0 * scores
    d2 = jnp.maximum(t, 0.0)
    d = jnp.sqrt(d2)
    dmin = jnp.min(d, axis=1, keepdims=True)
    iota = lax.broadcasted_iota(jnp.int32, (_ROWS, _K), 1)
    idx = jnp.min(jnp.where(d == dmin, iota, _K), axis=1, keepdims=True)
    tgt_ref[...] = idx
    onehot = (iota == idx).astype(jnp.float32)
    q = lax.dot_general(
        onehot, emb, (((1,), (0,)), ((), ())),
        preferred_element_type=jnp.float32,
        precision=lax.Precision.HIGHEST)
    q_ref[...] = q
    qst = x + (q - x)
    part = jnp.sum((x - qst) * (x - qst)).reshape(1, 1)
    @pl.when(i == 0)
    def _():
        loss_ref[...] = part
    @pl.when(i > 0)
    def _():
        loss_ref[...] = loss_ref[...] + part


def _tc_assign(flat, f2, e2, embeddings):
    n = flat.shape[0]
    grid = (n // _ROWS,)
    return pl.pallas_call(
        _tc_body,
        grid=grid,
        in_specs=[
            pl.BlockSpec((_ROWS, _D), lambda i: (i, 0)),
            pl.BlockSpec((_ROWS, 1), lambda i: (i, 0)),
            pl.BlockSpec((1, _K), lambda i: (0, 0)),
            pl.BlockSpec((_K, _D), lambda i: (0, 0)),
        ],
        out_specs=[
            pl.BlockSpec((_ROWS, 1), lambda i: (i, 0)),
            pl.BlockSpec((_ROWS, _D), lambda i: (i, 0)),
            pl.BlockSpec((1, 1), lambda i: (0, 0)),
        ],
        out_shape=[
            jax.ShapeDtypeStruct((n, 1), jnp.int32),
            jax.ShapeDtypeStruct((n, _D), jnp.float32),
            jax.ShapeDtypeStruct((1, 1), jnp.float32),
        ],
        compiler_params=pltpu.CompilerParams(
            dimension_semantics=("arbitrary",)),
    )(flat, f2, e2, embeddings)


def kernel(features, embeddings):
    B, T, D = features.shape
    flat = features.reshape(-1, D)
    f2 = jnp.sum(flat * flat, axis=1, keepdims=True)
    e2 = jnp.sum(embeddings * embeddings, axis=1)[None, :]
    tgt, quantized_flat, loss_sum = _tc_assign(flat, f2, e2, embeddings)
    quantized = quantized_flat.reshape(B, T, D)
    targets = tgt.reshape(B, T)
    extra_losses = (0.25 / (flat.shape[0] * D)) * loss_sum[0, 0]
    return quantized, targets, extra_losses


# speedup vs baseline: 1.4587x; 1.4587x over previous
import functools

import jax
import jax.numpy as jnp
from jax import lax
from jax.experimental import pallas as pl
from jax.experimental.pallas import tpu as pltpu
from jax.experimental.pallas import tpu_sc as plsc

_K = 1024
_D = 256
_ROWS = 256

_NC = 2
_NS = 16
_NW = _NC * _NS


def _tc_body(x_ref, f2_ref, e2_ref, emb_ref, tgt_ref, loss_ref):
    i = pl.program_id(0)
    x = x_ref[...]
    emb = emb_ref[...]
    scores = lax.dot_general(
        x, emb, (((1,), (1,)), ((), ())),
        preferred_element_type=jnp.float32)
    t = f2_ref[...] + e2_ref[...] - 2.0 * scores
    d2 = jnp.maximum(t, 0.0)
    d = jnp.sqrt(d2)
    dmin = jnp.min(d, axis=1, keepdims=True)
    iota = lax.broadcasted_iota(jnp.int32, (_ROWS, _K), 1)
    idx = jnp.min(jnp.where(d == dmin, iota, _K), axis=1, keepdims=True)
    tgt_ref[...] = idx
    d2min = jnp.min(d2, axis=1, keepdims=True)
    part = jnp.sum(d2min).reshape(1, 1)
    @pl.when(i == 0)
    def _():
        loss_ref[...] = part
    @pl.when(i > 0)
    def _():
        loss_ref[...] = loss_ref[...] + part


def _tc_assign(flat, f2, e2, embeddings):
    n = flat.shape[0]
    grid = (n // _ROWS,)
    return pl.pallas_call(
        _tc_body,
        grid=grid,
        in_specs=[
            pl.BlockSpec((_ROWS, _D), lambda i: (i, 0)),
            pl.BlockSpec((_ROWS, 1), lambda i: (i, 0)),
            pl.BlockSpec((1, _K), lambda i: (0, 0)),
            pl.BlockSpec((_K, _D), lambda i: (0, 0)),
        ],
        out_specs=[
            pl.BlockSpec((_ROWS, 1), lambda i: (i, 0)),
            pl.BlockSpec((1, 1), lambda i: (0, 0)),
        ],
        out_shape=[
            jax.ShapeDtypeStruct((n, 1), jnp.int32),
            jax.ShapeDtypeStruct((1, 1), jnp.float32),
        ],
        compiler_params=pltpu.CompilerParams(
            dimension_semantics=("arbitrary",)),
    )(flat, f2, e2, embeddings)


def _make_sc_gather(n):
    b_per_w = n // _NW
    chunk = 256
    mesh = plsc.VectorSubcoreMesh(core_axis_name="c", subcore_axis_name="s")

    @functools.partial(
        pl.kernel, mesh=mesh,
        out_type=jax.ShapeDtypeStruct((n, _D), jnp.float32),
        scratch_types=[
            pltpu.VMEM((chunk,), jnp.int32),
            pltpu.VMEM((chunk, _D), jnp.float32),
            pltpu.SemaphoreType.DMA,
        ],
    )
    def sc_gather(table_hbm, idx_hbm, out_hbm, idx_v, rows_v, sem):
        wid = lax.axis_index("s") * _NC + lax.axis_index("c")
        base = wid * b_per_w
        for c in range(b_per_w // chunk):
            off = base + c * chunk
            pltpu.sync_copy(idx_hbm.at[pl.ds(off, chunk)], idx_v)
            pltpu.async_copy(table_hbm.at[idx_v], rows_v, sem).wait()
            pltpu.sync_copy(rows_v, out_hbm.at[pl.ds(off, chunk)])

    return sc_gather


def kernel(features, embeddings):
    B, T, D = features.shape
    flat = features.reshape(-1, D)
    n = flat.shape[0]
    f2 = jnp.sum(flat * flat, axis=1, keepdims=True)
    e2 = jnp.sum(embeddings * embeddings, axis=1)[None, :]
    tgt, loss_sum = _tc_assign(flat, f2, e2, embeddings)
    quantized_flat = _make_sc_gather(n)(embeddings, tgt.reshape(n))
    quantized = quantized_flat.reshape(B, T, D)
    targets = tgt.reshape(B, T)
    extra_losses = (0.25 / (n * D)) * loss_sum[0, 0]
    return quantized, targets, extra_losses


# speedup vs baseline: 1.5066x; 1.0329x over previous
import functools

import jax
import jax.numpy as jnp
from jax import lax
from jax.experimental import pallas as pl
from jax.experimental.pallas import tpu as pltpu
from jax.experimental.pallas import tpu_sc as plsc

_K = 1024
_D = 256
_ROWS = 256

_NC = 2
_NS = 16
_NW = _NC * _NS


def _tc_body(x_ref, f2_ref, e2_ref, emb_ref, tgt_ref, loss_ref):
    i = pl.program_id(0)
    x = x_ref[...]
    emb = emb_ref[...]
    scores = lax.dot_general(
        x, emb, (((1,), (1,)), ((), ())),
        preferred_element_type=jnp.float32)
    t = f2_ref[...] + e2_ref[...] - 2.0 * scores
    d2 = jnp.maximum(t, 0.0)
    d = jnp.sqrt(d2)
    dmin = jnp.min(d, axis=1, keepdims=True)
    iota = lax.broadcasted_iota(
        jnp.int32, (_ROWS, _K), 1).astype(jnp.float32)
    idxf = jnp.min(jnp.where(d == dmin, iota, float(2 * _K)),
                   axis=1, keepdims=True)
    tgt_ref[...] = idxf.astype(jnp.int32)
    part = jnp.sum(dmin * dmin).reshape(1, 1)
    @pl.when(i == 0)
    def _():
        loss_ref[...] = part
    @pl.when(i > 0)
    def _():
        loss_ref[...] = loss_ref[...] + part


def _tc_assign(flat, f2, e2, embeddings):
    n = flat.shape[0]
    grid = (n // _ROWS,)
    return pl.pallas_call(
        _tc_body,
        grid=grid,
        in_specs=[
            pl.BlockSpec((_ROWS, _D), lambda i: (i, 0)),
            pl.BlockSpec((_ROWS, 1), lambda i: (i, 0)),
            pl.BlockSpec((1, _K), lambda i: (0, 0)),
            pl.BlockSpec((_K, _D), lambda i: (0, 0)),
        ],
        out_specs=[
            pl.BlockSpec((_ROWS, 1), lambda i: (i, 0)),
            pl.BlockSpec((1, 1), lambda i: (0, 0)),
        ],
        out_shape=[
            jax.ShapeDtypeStruct((n, 1), jnp.int32),
            jax.ShapeDtypeStruct((1, 1), jnp.float32),
        ],
        compiler_params=pltpu.CompilerParams(
            dimension_semantics=("arbitrary",)),
    )(flat, f2, e2, embeddings)


def _make_sc_gather(n):
    b_per_w = n // _NW
    chunk = 256
    mesh = plsc.VectorSubcoreMesh(core_axis_name="c", subcore_axis_name="s")

    @functools.partial(
        pl.kernel, mesh=mesh,
        out_type=jax.ShapeDtypeStruct((n, _D), jnp.float32),
        scratch_types=[
            pltpu.VMEM((chunk,), jnp.int32),
            pltpu.VMEM((chunk, _D), jnp.float32),
            pltpu.SemaphoreType.DMA,
        ],
    )
    def sc_gather(table_hbm, idx_hbm, out_hbm, idx_v, rows_v, sem):
        wid = lax.axis_index("s") * _NC + lax.axis_index("c")
        base = wid * b_per_w
        for c in range(b_per_w // chunk):
            off = base + c * chunk
            pltpu.sync_copy(idx_hbm.at[pl.ds(off, chunk)], idx_v)
            pltpu.async_copy(table_hbm.at[idx_v], rows_v, sem).wait()
            pltpu.sync_copy(rows_v, out_hbm.at[pl.ds(off, chunk)])

    return sc_gather


def kernel(features, embeddings):
    B, T, D = features.shape
    flat = features.reshape(-1, D)
    n = flat.shape[0]
    f2 = jnp.sum(flat * flat, axis=1, keepdims=True)
    e2 = jnp.sum(embeddings * embeddings, axis=1)[None, :]
    tgt, loss_sum = _tc_assign(flat, f2, e2, embeddings)
    quantized_flat = _make_sc_gather(n)(embeddings, tgt.reshape(n))
    quantized = quantized_flat.reshape(B, T, D)
    targets = tgt.reshape(B, T)
    extra_losses = (0.25 / (n * D)) * loss_sum[0, 0]
    return quantized, targets, extra_losses


# speedup vs baseline: 1.7752x; 1.1782x over previous
import functools

import jax
import jax.numpy as jnp
from jax import lax
from jax.experimental import pallas as pl
from jax.experimental.pallas import tpu as pltpu
from jax.experimental.pallas import tpu_sc as plsc

_K = 1024
_D = 256
_ROWS = 512

_NC = 2
_NS = 16
_NW = _NC * _NS


def _tc_body(x_ref, f2_ref, e2_ref, emb_ref, tgt_ref, loss_ref):
    i = pl.program_id(0)
    x = x_ref[...]
    emb = emb_ref[...]
    scores = lax.dot_general(
        x, emb, (((1,), (1,)), ((), ())),
        preferred_element_type=jnp.float32)
    t = f2_ref[...] + e2_ref[...] - 2.0 * scores
    d2 = jnp.maximum(t, 0.0)
    d = jnp.sqrt(d2)
    dmin = jnp.min(d, axis=1, keepdims=True)
    iota = lax.broadcasted_iota(
        jnp.int32, (_ROWS, _K), 1).astype(jnp.float32)
    idxf = jnp.min(jnp.where(d == dmin, iota, float(2 * _K)),
                   axis=1, keepdims=True)
    tgt_ref[...] = idxf.astype(jnp.int32)
    part = jnp.sum(dmin * dmin).reshape(1, 1)
    @pl.when(i == 0)
    def _():
        loss_ref[...] = part
    @pl.when(i > 0)
    def _():
        loss_ref[...] = loss_ref[...] + part


def _tc_assign(flat, f2, e2, embeddings):
    n = flat.shape[0]
    grid = (n // _ROWS,)
    return pl.pallas_call(
        _tc_body,
        grid=grid,
        in_specs=[
            pl.BlockSpec((_ROWS, _D), lambda i: (i, 0)),
            pl.BlockSpec((_ROWS, 1), lambda i: (i, 0)),
            pl.BlockSpec((1, _K), lambda i: (0, 0)),
            pl.BlockSpec((_K, _D), lambda i: (0, 0)),
        ],
        out_specs=[
            pl.BlockSpec((_ROWS, 1), lambda i: (i, 0)),
            pl.BlockSpec((1, 1), lambda i: (0, 0)),
        ],
        out_shape=[
            jax.ShapeDtypeStruct((n, 1), jnp.int32),
            jax.ShapeDtypeStruct((1, 1), jnp.float32),
        ],
        compiler_params=pltpu.CompilerParams(
            dimension_semantics=("arbitrary",)),
    )(flat, f2, e2, embeddings)


def _make_sc_gather(n):
    b_per_w = n // _NW
    chunk = 256
    mesh = plsc.VectorSubcoreMesh(core_axis_name="c", subcore_axis_name="s")

    @functools.partial(
        pl.kernel, mesh=mesh,
        out_type=jax.ShapeDtypeStruct((n, _D), jnp.float32),
        scratch_types=[
            pltpu.VMEM((chunk,), jnp.int32),
            pltpu.VMEM((chunk, _D), jnp.float32),
            pltpu.SemaphoreType.DMA,
        ],
    )
    def sc_gather(table_hbm, idx_hbm, out_hbm, idx_v, rows_v, sem):
        wid = lax.axis_index("s") * _NC + lax.axis_index("c")
        base = wid * b_per_w
        for c in range(b_per_w // chunk):
            off = base + c * chunk
            pltpu.sync_copy(idx_hbm.at[pl.ds(off, chunk)], idx_v)
            pltpu.async_copy(table_hbm.at[idx_v], rows_v, sem).wait()
            pltpu.sync_copy(rows_v, out_hbm.at[pl.ds(off, chunk)])

    return sc_gather


def kernel(features, embeddings):
    B, T, D = features.shape
    flat = features.reshape(-1, D)
    n = flat.shape[0]
    f2 = jnp.sum(flat * flat, axis=1, keepdims=True)
    e2 = jnp.sum(embeddings * embeddings, axis=1)[None, :]
    tgt, loss_sum = _tc_assign(flat, f2, e2, embeddings)
    quantized_flat = _make_sc_gather(n)(embeddings, tgt.reshape(n))
    quantized = quantized_flat.reshape(B, T, D)
    targets = tgt.reshape(B, T)
    extra_losses = (0.25 / (n * D)) * loss_sum[0, 0]
    return quantized, targets, extra_losses


# speedup vs baseline: 1.8903x; 1.0649x over previous
import functools

import jax
import jax.numpy as jnp
from jax import lax
from jax.experimental import pallas as pl
from jax.experimental.pallas import tpu as pltpu
from jax.experimental.pallas import tpu_sc as plsc

_K = 1024
_D = 256
_ROWS = 1024

_NC = 2
_NS = 16
_NW = _NC * _NS


def _tc_body(x_ref, f2_ref, e2_ref, emb_ref, tgt_ref, loss_ref):
    i = pl.program_id(0)
    x = x_ref[...]
    emb = emb_ref[...]
    scores = lax.dot_general(
        x, emb, (((1,), (1,)), ((), ())),
        preferred_element_type=jnp.float32)
    t = f2_ref[...] + e2_ref[...] - 2.0 * scores
    d2 = jnp.maximum(t, 0.0)
    d = jnp.sqrt(d2)
    dmin = jnp.min(d, axis=1, keepdims=True)
    iota = lax.broadcasted_iota(
        jnp.int32, (_ROWS, _K), 1).astype(jnp.float32)
    idxf = jnp.min(jnp.where(d == dmin, iota, float(2 * _K)),
                   axis=1, keepdims=True)
    tgt_ref[...] = idxf.astype(jnp.int32)
    part = jnp.sum(dmin * dmin).reshape(1, 1)
    @pl.when(i == 0)
    def _():
        loss_ref[...] = part
    @pl.when(i > 0)
    def _():
        loss_ref[...] = loss_ref[...] + part


def _tc_assign(flat, f2, e2, embeddings):
    n = flat.shape[0]
    grid = (n // _ROWS,)
    return pl.pallas_call(
        _tc_body,
        grid=grid,
        in_specs=[
            pl.BlockSpec((_ROWS, _D), lambda i: (i, 0)),
            pl.BlockSpec((_ROWS, 1), lambda i: (i, 0)),
            pl.BlockSpec((1, _K), lambda i: (0, 0)),
            pl.BlockSpec((_K, _D), lambda i: (0, 0)),
        ],
        out_specs=[
            pl.BlockSpec((_ROWS, 1), lambda i: (i, 0)),
            pl.BlockSpec((1, 1), lambda i: (0, 0)),
        ],
        out_shape=[
            jax.ShapeDtypeStruct((n, 1), jnp.int32),
            jax.ShapeDtypeStruct((1, 1), jnp.float32),
        ],
        compiler_params=pltpu.CompilerParams(
            dimension_semantics=("arbitrary",)),
    )(flat, f2, e2, embeddings)


def _make_sc_gather(n):
    b_per_w = n // _NW
    chunk = 256
    mesh = plsc.VectorSubcoreMesh(core_axis_name="c", subcore_axis_name="s")

    @functools.partial(
        pl.kernel, mesh=mesh,
        out_type=jax.ShapeDtypeStruct((n, _D), jnp.float32),
        scratch_types=[
            pltpu.VMEM((chunk,), jnp.int32),
            pltpu.VMEM((chunk, _D), jnp.float32),
            pltpu.SemaphoreType.DMA,
        ],
    )
    def sc_gather(table_hbm, idx_hbm, out_hbm, idx_v, rows_v, sem):
        wid = lax.axis_index("s") * _NC + lax.axis_index("c")
        base = wid * b_per_w
        for c in range(b_per_w // chunk):
            off = base + c * chunk
            pltpu.sync_copy(idx_hbm.at[pl.ds(off, chunk)], idx_v)
            pltpu.async_copy(table_hbm.at[idx_v], rows_v, sem).wait()
            pltpu.sync_copy(rows_v, out_hbm.at[pl.ds(off, chunk)])

    return sc_gather


def kernel(features, embeddings):
    B, T, D = features.shape
    flat = features.reshape(-1, D)
    n = flat.shape[0]
    f2 = jnp.sum(flat * flat, axis=1, keepdims=True)
    e2 = jnp.sum(embeddings * embeddings, axis=1)[None, :]
    tgt, loss_sum = _tc_assign(flat, f2, e2, embeddings)
    quantized_flat = _make_sc_gather(n)(embeddings, tgt.reshape(n))
    quantized = quantized_flat.reshape(B, T, D)
    targets = tgt.reshape(B, T)
    extra_losses = (0.25 / (n * D)) * loss_sum[0, 0]
    return quantized, targets, extra_losses


# speedup vs baseline: 1.9341x; 1.0232x over previous
import functools

import jax
import jax.numpy as jnp
from jax import lax
from jax.experimental import pallas as pl
from jax.experimental.pallas import tpu as pltpu
from jax.experimental.pallas import tpu_sc as plsc

_K = 1024
_D = 256
_ROWS = 2048

_NC = 2
_NS = 16
_NW = _NC * _NS


def _tc_body(x_ref, f2_ref, e2_ref, emb_ref, tgt_ref, loss_ref):
    i = pl.program_id(0)
    x = x_ref[...]
    emb = emb_ref[...]
    scores = lax.dot_general(
        x, emb, (((1,), (1,)), ((), ())),
        preferred_element_type=jnp.float32)
    t = f2_ref[...] + e2_ref[...] - 2.0 * scores
    d2 = jnp.maximum(t, 0.0)
    d = jnp.sqrt(d2)
    dmin = jnp.min(d, axis=1, keepdims=True)
    iota = lax.broadcasted_iota(
        jnp.int32, (_ROWS, _K), 1).astype(jnp.float32)
    idxf = jnp.min(jnp.where(d == dmin, iota, float(2 * _K)),
                   axis=1, keepdims=True)
    tgt_ref[...] = idxf.astype(jnp.int32)
    part = jnp.sum(dmin * dmin).reshape(1, 1)
    @pl.when(i == 0)
    def _():
        loss_ref[...] = part
    @pl.when(i > 0)
    def _():
        loss_ref[...] = loss_ref[...] + part


def _tc_assign(flat, f2, e2, embeddings):
    n = flat.shape[0]
    grid = (n // _ROWS,)
    return pl.pallas_call(
        _tc_body,
        grid=grid,
        in_specs=[
            pl.BlockSpec((_ROWS, _D), lambda i: (i, 0)),
            pl.BlockSpec((_ROWS, 1), lambda i: (i, 0)),
            pl.BlockSpec((1, _K), lambda i: (0, 0)),
            pl.BlockSpec((_K, _D), lambda i: (0, 0)),
        ],
        out_specs=[
            pl.BlockSpec((_ROWS, 1), lambda i: (i, 0)),
            pl.BlockSpec((1, 1), lambda i: (0, 0)),
        ],
        out_shape=[
            jax.ShapeDtypeStruct((n, 1), jnp.int32),
            jax.ShapeDtypeStruct((1, 1), jnp.float32),
        ],
        compiler_params=pltpu.CompilerParams(
            dimension_semantics=("arbitrary",)),
    )(flat, f2, e2, embeddings)


def _make_sc_gather(n):
    b_per_w = n // _NW
    chunk = 256
    mesh = plsc.VectorSubcoreMesh(core_axis_name="c", subcore_axis_name="s")

    @functools.partial(
        pl.kernel, mesh=mesh,
        out_type=jax.ShapeDtypeStruct((n, _D), jnp.float32),
        scratch_types=[
            pltpu.VMEM((chunk,), jnp.int32),
            pltpu.VMEM((chunk, _D), jnp.float32),
            pltpu.SemaphoreType.DMA,
        ],
    )
    def sc_gather(table_hbm, idx_hbm, out_hbm, idx_v, rows_v, sem):
        wid = lax.axis_index("s") * _NC + lax.axis_index("c")
        base = wid * b_per_w
        for c in range(b_per_w // chunk):
            off = base + c * chunk
            pltpu.sync_copy(idx_hbm.at[pl.ds(off, chunk)], idx_v)
            pltpu.async_copy(table_hbm.at[idx_v], rows_v, sem).wait()
            pltpu.sync_copy(rows_v, out_hbm.at[pl.ds(off, chunk)])

    return sc_gather


def kernel(features, embeddings):
    B, T, D = features.shape
    flat = features.reshape(-1, D)
    n = flat.shape[0]
    f2 = jnp.sum(flat * flat, axis=1, keepdims=True)
    e2 = jnp.sum(embeddings * embeddings, axis=1)[None, :]
    tgt, loss_sum = _tc_assign(flat, f2, e2, embeddings)
    quantized_flat = _make_sc_gather(n)(embeddings, tgt.reshape(n))
    quantized = quantized_flat.reshape(B, T, D)
    targets = tgt.reshape(B, T)
    extra_losses = (0.25 / (n * D)) * loss_sum[0, 0]
    return quantized, targets, extra_losses
